# baseline (speedup 1.0000x reference)
import functools

import jax
import jax.numpy as jnp
from jax import lax
from jax.experimental import pallas as pl
from jax.experimental.pallas import tpu as pltpu
from jax.experimental.pallas import tpu_sc as plsc

N = 10000
E = 320000
D_IN = 128
D_EDGE = 16
D_OUT = 16
D_HID = 32
N_STEPS = 2

NC = 2
NS = 16
NW = NC * NS

EB = 2000
EPW = E // NW
CH = 2000
EPC = E // NC

_PREC = lax.Precision.HIGHEST


def _h0_body(nf_ref, w_ref, b_ref, o_ref):
    o_ref[...] = jnp.maximum(
        jnp.dot(nf_ref[...], w_ref[...], precision=_PREC) + b_ref[...], 0.0)


def _project_nodes(node_feats, proj_W, proj_b):
    return pl.pallas_call(
        _h0_body,
        out_shape=jax.ShapeDtypeStruct((N, D_OUT), jnp.float32),
    )(node_feats, proj_W, proj_b.reshape(1, D_OUT))


PK = 8
PB = 1000
KW = PK * D_HID * D_OUT


def _msg_body(ef_ref, hs_ref, e1wt_ref, e1bt_ref, w2rt_ref, bbt_ref, m_ref):
    efT = ef_ref[...].T
    hT = hs_ref[...].T
    e1wt = e1wt_ref[...]
    e1bt = e1bt_ref[...]
    w2rt = w2rt_ref[...]
    bbt = bbt_ref[...]
    parts = []
    for j in range(PK):
        efj = efT[j * D_EDGE:(j + 1) * D_EDGE, :]
        hj = hT[j * D_OUT:(j + 1) * D_OUT, :]
        zj = jnp.maximum(
            jnp.dot(e1wt, efj, preferred_element_type=jnp.float32,
                    precision=_PREC) + e1bt,
            0.0)
        pj = (zj[:, None, :] * hj[None, :, :]).reshape(D_HID * D_OUT, PB)
        mj = (jnp.dot(w2rt, pj, preferred_element_type=jnp.float32,
                      precision=_PREC)
              + jnp.dot(bbt, hj, preferred_element_type=jnp.float32,
                        precision=_PREC))
        parts.append(mj)
    mT = jnp.concatenate(parts, axis=0)
    m_ref[...] = mT.T


def _messages(efP, hP, e1_WT, e1_bT, W2rT, BbT):
    grid = (E // PK // PB,)
    return pl.pallas_call(
        _msg_body,
        grid=grid,
        in_specs=[
            pl.BlockSpec((PB, PK * D_EDGE), lambda i: (i, 0)),
            pl.BlockSpec((PB, PK * D_OUT), lambda i: (i, 0)),
            pl.BlockSpec((D_HID, D_EDGE), lambda i: (0, 0)),
            pl.BlockSpec((D_HID, 1), lambda i: (0, 0)),
            pl.BlockSpec((D_OUT, D_HID * D_OUT), lambda i: (0, 0)),
            pl.BlockSpec((D_OUT, D_OUT), lambda i: (0, 0)),
        ],
        out_specs=pl.BlockSpec((PB, PK * D_OUT), lambda i: (i, 0)),
        out_shape=jax.ShapeDtypeStruct((E // PK, PK * D_OUT), jnp.float32),
    )(efP, hP, e1_WT, e1_bT, W2rT, BbT)


def _combine_body(p_ref, b_ref, o_ref):
    o_ref[...] = jnp.maximum(p_ref[0] + p_ref[1] + b_ref[...], 0.0)


def _combine(partials, conv_b):
    return pl.pallas_call(
        _combine_body,
        out_shape=jax.ShapeDtypeStruct((N, D_OUT), jnp.float32),
    )(partials, conv_b.reshape(1, D_OUT))


def _final_body(p_ref, b_ref, pw_ref, pb_ref, o_ref):
    h = jnp.maximum(p_ref[0] + p_ref[1] + b_ref[...], 0.0)
    g = jnp.mean(h, axis=0, keepdims=True)
    o_ref[...] = jnp.dot(g, pw_ref[...], precision=_PREC) + pb_ref[...]


def _final(partials, conv_b, pred_W, pred_b):
    return pl.pallas_call(
        _final_body,
        out_shape=jax.ShapeDtypeStruct((1, pred_W.shape[1]), jnp.float32),
    )(partials, conv_b.reshape(1, D_OUT), pred_W,
      pred_b.reshape(1, pred_W.shape[1]))


_MESH = dict(core_axis_name="c", subcore_axis_name="s", num_cores=NC,
             num_subcores=NS)
_SC_PARAMS = pltpu.CompilerParams(use_tc_tiling_on_sc=False)


def _gather_kernel_body(h_hbm, src_hbm, out_hbm, idx_v, rows_v, sem):
    wid = lax.axis_index("s") * NC + lax.axis_index("c")

    def body(i, carry):
        base = wid * EPW + i * CH
        pltpu.sync_copy(src_hbm.at[pl.ds(base, CH)], idx_v)
        pltpu.async_copy(h_hbm.at[idx_v], rows_v, sem).wait()
        pltpu.sync_copy(rows_v, out_hbm.at[pl.ds(base, CH)])
        return carry

    lax.fori_loop(0, EPW // CH, body, 0)


def _sc_gather(h, src):
    k = functools.partial(
        pl.kernel,
        out_type=jax.ShapeDtypeStruct((E, D_OUT), jnp.float32),
        mesh=plsc.VectorSubcoreMesh(**_MESH),
        compiler_params=_SC_PARAMS,
        scratch_types=[
            pltpu.VMEM((CH,), jnp.int32),
            pltpu.VMEM((CH, D_OUT), jnp.float32),
            pltpu.SemaphoreType.DMA,
        ],
    )(_gather_kernel_body)
    return k(h, src)


def _scatter_kernel_body(m_hbm, dst_hbm, zero_hbm, out_hbm, idx_v, rows_v,
                         acc_sh, sem):
    cid = lax.axis_index("c")
    sid = lax.axis_index("s")

    @pl.when(sid == 0)
    def _():
        pltpu.sync_copy(zero_hbm, acc_sh)

    plsc.subcore_barrier()

    def body(i, carry):
        base = cid * EPC + sid * EPW + i * CH
        pltpu.sync_copy(dst_hbm.at[pl.ds(base, CH)], idx_v)
        pltpu.sync_copy(m_hbm.at[pl.ds(base, CH)], rows_v)
        pltpu.sync_copy(rows_v, acc_sh.at[idx_v], add=True)
        return carry

    lax.fori_loop(0, EPW // CH, body, 0)

    plsc.subcore_barrier()

    rows = N // NS
    pltpu.sync_copy(acc_sh.at[pl.ds(sid * rows, rows)],
                    out_hbm.at[cid].at[pl.ds(sid * rows, rows)])


def _sc_scatter(m, dst):
    zeros = jnp.zeros((N, D_OUT), jnp.float32)
    k = functools.partial(
        pl.kernel,
        out_type=jax.ShapeDtypeStruct((NC, N, D_OUT), jnp.float32),
        mesh=plsc.VectorSubcoreMesh(**_MESH),
        compiler_params=_SC_PARAMS,
        scratch_types=[
            pltpu.VMEM((CH,), jnp.int32),
            pltpu.VMEM((CH, D_OUT), jnp.float32),
            pltpu.VMEM_SHARED((N, D_OUT), jnp.float32),
            pltpu.SemaphoreType.DMA,
        ],
    )(_scatter_kernel_body)
    return k(m, dst, zeros)


def kernel(node_feats, edge_feats, edge_index, proj_W, proj_b, e1_W, e1_b,
           e2_W, e2_b, conv_b, pred_W, pred_b):
    src = edge_index[0]
    dst = edge_index[1]
    f32 = jnp.float32
    bf16 = jnp.bfloat16
    W2r = e2_W.reshape(D_HID, D_OUT, D_OUT).reshape(D_HID * D_OUT, D_OUT)
    Bb = e2_b.reshape(D_OUT, D_OUT)
    e1_WT = e1_W.T
    e1_bT = e1_b.reshape(D_HID, 1)
    W2rT = W2r.T
    BbT = Bb.T
    efP = edge_feats.reshape(E // PK, PK * D_EDGE)

    h = _project_nodes(node_feats, proj_W, proj_b)
    for step in range(N_STEPS):
        h_src = _sc_gather(h, src)
        hP = h_src.reshape(E // PK, PK * D_OUT)
        mP = _messages(efP, hP, e1_WT, e1_bT, W2rT, BbT)
        partials = _sc_scatter(mP.reshape(E, D_OUT), dst)
        if step < N_STEPS - 1:
            h = _combine(partials, conv_b)
        else:
            out = _final(partials, conv_b, pred_W, pred_b)
    return out


# speedup vs baseline: 1.7894x; 1.7894x over previous
import functools

import jax
import jax.numpy as jnp
from jax import lax
from jax.experimental import pallas as pl
from jax.experimental.pallas import tpu as pltpu
from jax.experimental.pallas import tpu_sc as plsc

N = 10000
E = 320000
D_IN = 128
D_EDGE = 16
D_OUT = 16
D_HID = 32
N_STEPS = 2

NC = 2
NS = 16
NW = NC * NS

EB = 2000
EPW = E // NW
CH = 2000
EPC = E // NC

_PREC = lax.Precision.HIGHEST
_PREC_MSG = lax.Precision.HIGH


def _h0_body(nf_ref, w_ref, b_ref, o_ref):
    o_ref[...] = jnp.maximum(
        jnp.dot(nf_ref[...], w_ref[...], precision=_PREC) + b_ref[...], 0.0)


def _project_nodes(node_feats, proj_W, proj_b):
    return pl.pallas_call(
        _h0_body,
        out_shape=jax.ShapeDtypeStruct((N, D_OUT), jnp.float32),
    )(node_feats, proj_W, proj_b.reshape(1, D_OUT))


PK = 8
PB = 1000
KW = PK * D_HID * D_OUT


def _bdot(a, b):
    return jnp.dot(a, b, preferred_element_type=jnp.float32)


def _split(x):
    hi = x.astype(jnp.bfloat16)
    lo = (x - hi.astype(jnp.float32)).astype(jnp.bfloat16)
    return hi, lo


def _msg_body(ef_ref, hs_ref, e1wt_ref, e1bt_ref, w2rt_ref, bbt_ref, m_ref):
    efT = ef_ref[...].T
    hT = hs_ref[...].T
    e1h = e1wt_ref[...][:, :D_EDGE]
    e1l = e1wt_ref[...][:, D_EDGE:]
    e1bt = e1bt_ref[...]
    w2h = w2rt_ref[...][:D_OUT]
    w2l = w2rt_ref[...][D_OUT:]
    bbt = bbt_ref[...]
    parts = []
    for j in range(PK):
        efh, efl = _split(efT[j * D_EDGE:(j + 1) * D_EDGE, :])
        hj = hT[j * D_OUT:(j + 1) * D_OUT, :]
        zj = jnp.maximum(
            _bdot(e1h, efh) + _bdot(e1l, efh) + _bdot(e1h, efl) + e1bt, 0.0)
        pj = (zj[:, None, :] * hj[None, :, :]).reshape(D_HID * D_OUT, PB)
        ph, plo = _split(pj)
        mj = (_bdot(w2h, ph) + _bdot(w2l, ph) + _bdot(w2h, plo)
              + _bdot(bbt, hj.astype(jnp.bfloat16)))
        parts.append(mj)
    mT = jnp.concatenate(parts, axis=0)
    m_ref[...] = mT.T


def _messages(efP, hP, e1_WT, e1_bT, W2rT, BbT):
    grid = (E // PK // PB,)
    return pl.pallas_call(
        _msg_body,
        grid=grid,
        in_specs=[
            pl.BlockSpec((PB, PK * D_EDGE), lambda i: (i, 0)),
            pl.BlockSpec((PB, PK * D_OUT), lambda i: (i, 0)),
            pl.BlockSpec((D_HID, 2 * D_EDGE), lambda i: (0, 0)),
            pl.BlockSpec((D_HID, 1), lambda i: (0, 0)),
            pl.BlockSpec((2 * D_OUT, D_HID * D_OUT), lambda i: (0, 0)),
            pl.BlockSpec((D_OUT, D_OUT), lambda i: (0, 0)),
        ],
        out_specs=pl.BlockSpec((PB, PK * D_OUT), lambda i: (i, 0)),
        out_shape=jax.ShapeDtypeStruct((E // PK, PK * D_OUT), jnp.float32),
    )(efP, hP, e1_WT, e1_bT, W2rT, BbT)


def _combine_body(p_ref, b_ref, o_ref):
    o_ref[...] = jnp.maximum(p_ref[0] + p_ref[1] + b_ref[...], 0.0)


def _combine(partials, conv_b):
    return pl.pallas_call(
        _combine_body,
        out_shape=jax.ShapeDtypeStruct((N, D_OUT), jnp.float32),
    )(partials, conv_b.reshape(1, D_OUT))


def _final_body(p_ref, b_ref, pw_ref, pb_ref, o_ref):
    h = jnp.maximum(p_ref[0] + p_ref[1] + b_ref[...], 0.0)
    g = jnp.mean(h, axis=0, keepdims=True)
    o_ref[...] = jnp.dot(g, pw_ref[...], precision=_PREC) + pb_ref[...]


def _final(partials, conv_b, pred_W, pred_b):
    return pl.pallas_call(
        _final_body,
        out_shape=jax.ShapeDtypeStruct((1, pred_W.shape[1]), jnp.float32),
    )(partials, conv_b.reshape(1, D_OUT), pred_W,
      pred_b.reshape(1, pred_W.shape[1]))


_MESH = dict(core_axis_name="c", subcore_axis_name="s", num_cores=NC,
             num_subcores=NS)
_SC_PARAMS = pltpu.CompilerParams(use_tc_tiling_on_sc=False)


def _gather_kernel_body(h_hbm, src_hbm, out_hbm, idx_v, rows_v, sem):
    wid = lax.axis_index("s") * NC + lax.axis_index("c")

    def body(i, carry):
        base = wid * EPW + i * CH
        pltpu.sync_copy(src_hbm.at[pl.ds(base, CH)], idx_v)
        pltpu.async_copy(h_hbm.at[idx_v], rows_v, sem).wait()
        pltpu.sync_copy(rows_v, out_hbm.at[pl.ds(base, CH)])
        return carry

    lax.fori_loop(0, EPW // CH, body, 0)


def _sc_gather(h, src):
    k = functools.partial(
        pl.kernel,
        out_type=jax.ShapeDtypeStruct((E, D_OUT), jnp.float32),
        mesh=plsc.VectorSubcoreMesh(**_MESH),
        compiler_params=_SC_PARAMS,
        scratch_types=[
            pltpu.VMEM((CH,), jnp.int32),
            pltpu.VMEM((CH, D_OUT), jnp.float32),
            pltpu.SemaphoreType.DMA,
        ],
    )(_gather_kernel_body)
    return k(h, src)


def _scatter_kernel_body(m_hbm, dst_hbm, zero_hbm, out_hbm, idx_v, rows_v,
                         acc_sh, sem):
    cid = lax.axis_index("c")
    sid = lax.axis_index("s")

    @pl.when(sid == 0)
    def _():
        pltpu.sync_copy(zero_hbm, acc_sh)

    plsc.subcore_barrier()

    def body(i, carry):
        base = cid * EPC + sid * EPW + i * CH
        pltpu.sync_copy(dst_hbm.at[pl.ds(base, CH)], idx_v)
        pltpu.sync_copy(m_hbm.at[pl.ds(base, CH)], rows_v)
        pltpu.sync_copy(rows_v, acc_sh.at[idx_v], add=True)
        return carry

    lax.fori_loop(0, EPW // CH, body, 0)

    plsc.subcore_barrier()

    rows = N // NS
    pltpu.sync_copy(acc_sh.at[pl.ds(sid * rows, rows)],
                    out_hbm.at[cid].at[pl.ds(sid * rows, rows)])


def _sc_scatter(m, dst):
    zeros = jnp.zeros((N, D_OUT), jnp.float32)
    k = functools.partial(
        pl.kernel,
        out_type=jax.ShapeDtypeStruct((NC, N, D_OUT), jnp.float32),
        mesh=plsc.VectorSubcoreMesh(**_MESH),
        compiler_params=_SC_PARAMS,
        scratch_types=[
            pltpu.VMEM((CH,), jnp.int32),
            pltpu.VMEM((CH, D_OUT), jnp.float32),
            pltpu.VMEM_SHARED((N, D_OUT), jnp.float32),
            pltpu.SemaphoreType.DMA,
        ],
    )(_scatter_kernel_body)
    return k(m, dst, zeros)


def kernel(node_feats, edge_feats, edge_index, proj_W, proj_b, e1_W, e1_b,
           e2_W, e2_b, conv_b, pred_W, pred_b):
    src = edge_index[0]
    dst = edge_index[1]
    f32 = jnp.float32
    bf16 = jnp.bfloat16
    W2r = e2_W.reshape(D_HID, D_OUT, D_OUT).reshape(D_HID * D_OUT, D_OUT)
    Bb = e2_b.reshape(D_OUT, D_OUT)
    def _hilo(x):
        hi = x.astype(bf16)
        lo = (x - hi.astype(f32)).astype(bf16)
        return hi, lo

    e1h, e1l = _hilo(e1_W.T)
    e1_WT = jnp.concatenate([e1h, e1l], axis=1)
    e1_bT = e1_b.reshape(D_HID, 1)
    w2h, w2l = _hilo(W2r.T)
    W2rT = jnp.concatenate([w2h, w2l], axis=0)
    BbT = Bb.T.astype(bf16)
    efP = edge_feats.reshape(E // PK, PK * D_EDGE)

    h = _project_nodes(node_feats, proj_W, proj_b)
    for step in range(N_STEPS):
        h_src = _sc_gather(h, src)
        hP = h_src.reshape(E // PK, PK * D_OUT)
        mP = _messages(efP, hP, e1_WT, e1_bT, W2rT, BbT)
        partials = _sc_scatter(mP.reshape(E, D_OUT), dst)
        if step < N_STEPS - 1:
            h = _combine(partials, conv_b)
        else:
            out = _final(partials, conv_b, pred_W, pred_b)
    return out
